# baseline (speedup 1.0000x reference)
import jax
import jax.numpy as jnp
from jax.experimental import pallas as pl
from jax.experimental.pallas import tpu as pltpu

_BN = 512
_SUB = 256


def _layer_kernel(a0_ref, a1_ref, xh_ref, xl_ref, nrm_ref, watt_ref,
                  w_ref, b_ref, o_ref):
    xh = xh_ref[...]
    xl = xl_ref[...]
    watt = watt_ref[...]
    w = w_ref[...]
    bias = b_ref[...]
    bn = o_ref.shape[0]

    def split_dot(a_ref, lo):
        a = a_ref[0, lo:lo + _SUB, :]
        ah = a.astype(jnp.bfloat16)
        al = (a - ah.astype(jnp.float32)).astype(jnp.bfloat16)
        return (jnp.dot(ah, xh, preferred_element_type=jnp.float32)
                + jnp.dot(ah, xl, preferred_element_type=jnp.float32)
                + jnp.dot(al, xh, preferred_element_type=jnp.float32))

    for h in range(bn // _SUB):
        lo = h * _SUB
        t0 = split_dot(a0_ref, lo)
        t1 = split_dot(a1_ref, lo)
        nrm = nrm_ref[lo:lo + _SUB]
        t0 = t0 / nrm[:, 0:1]
        t1 = t1 / nrm[:, 1:2]
        u = t0 - t1
        d = jnp.dot(u, watt, preferred_element_type=jnp.float32)
        p = 1.0 / (1.0 + jnp.exp(-d))
        comb = t1 + p * u
        out = jnp.dot(comb, w, preferred_element_type=jnp.float32)
        out = jnp.maximum(out + bias, 0.0)
        o_ref[lo:lo + _SUB, :] = out


def _split_bf16(x):
    hi = x.astype(jnp.bfloat16)
    lo = (x - hi.astype(jnp.float32)).astype(jnp.bfloat16)
    return hi, lo


def _layer(xh, xl, motifs_all, nrm_t, w_att, w, b, *, interpret=False):
    n = xh.shape[0]
    d_in = xh.shape[1]
    d_out = w.shape[1]
    m = nrm_t.shape[1]
    bn = _BN
    grid = (n // bn,)
    return pl.pallas_call(
        _layer_kernel,
        grid=grid,
        in_specs=[
            pl.BlockSpec((1, bn, n), lambda i: (0, i, 0)),
            pl.BlockSpec((1, bn, n), lambda i: (1, i, 0)),
            pl.BlockSpec((n, d_in), lambda i: (0, 0)),
            pl.BlockSpec((n, d_in), lambda i: (0, 0)),
            pl.BlockSpec((bn, m), lambda i: (i, 0)),
            pl.BlockSpec((d_in, 1), lambda i: (0, 0)),
            pl.BlockSpec((d_in, d_out), lambda i: (0, 0)),
            pl.BlockSpec((1, d_out), lambda i: (0, 0)),
        ],
        out_specs=pl.BlockSpec((bn, d_out), lambda i: (i, 0)),
        out_shape=jax.ShapeDtypeStruct((n, d_out), jnp.float32),
        compiler_params=pltpu.CompilerParams(
            dimension_semantics=("arbitrary",)),
        interpret=interpret,
    )(motifs_all, motifs_all, xh, xl, nrm_t, w_att, w, b)


@jax.jit
def kernel(x, motifs_all, motifs_num, w_att0, b_att0, W0, b0,
           w_att1, b_att1, W1, b1):
    del b_att0, b_att1
    nrm_t = motifs_num.T
    xh, xl = _split_bf16(x)
    h = _layer(xh, xl, motifs_all, nrm_t,
               w_att0, W0, b0.reshape(1, -1))
    hh, hl = _split_bf16(h)
    return _layer(hh, hl, motifs_all, nrm_t,
                  w_att1, W1, b1.reshape(1, -1))
